# baseline (speedup 1.0000x reference)
import jax
import jax.numpy as jnp
from jax.experimental import pallas as pl
from jax.experimental.pallas import tpu as pltpu

N = 10000
HID = 64
NJ = 100
NQ = 32
HQ = 256
BLK = 80
R = N // BLK
D = 4
CB = 18
CLO = R - D - CB


def _block_of(j):
    phase1 = j < R
    s = jnp.where(phase1, j, j - R)
    return jnp.where(phase1, s, R - 1 - s)


def _need_fetch(j):
    return jnp.logical_and(j < 2 * R,
                           jnp.logical_or(j < R, _block_of(j) < CLO))


def _bn_relu(x, g, b):
    m = jnp.mean(x, axis=0, keepdims=True)
    d = x - m
    v = jnp.mean(d * d, axis=0, keepdims=True)
    return jnp.maximum(d / jnp.sqrt(v + 1e-5) * g + b, 0.0)


def _body(adj_ref, f_ref, W1_ref, b1_ref, gm_ref, bm_ref, W2_ref, b2_ref,
          go_ref, bo_ref, gp_ref, cand_ref, Wq0_ref, bq0_ref, Wq1_ref,
          bq1_ref, Wq2_ref, bq2_ref, Wq3_ref, bq3_ref, q_ref,
          ybuf, hbuf, hwb, cfbuf, buf, cache, sems):
    i = pl.program_id(0)
    b = _block_of(i)
    slot = jax.lax.rem(b, D)

    def _issue(blk):
        pltpu.make_async_copy(
            adj_ref.at[pl.ds(blk * BLK, BLK), :],
            buf.at[jax.lax.rem(blk, D)],
            sems.at[jax.lax.rem(blk, D)],
        ).start()

    @pl.when(i == 0)
    def _init():
        hwb[...] = jnp.dot(f_ref[...], W1_ref[0].astype(jnp.bfloat16),
                           preferred_element_type=jnp.float32
                           ).astype(jnp.bfloat16)
        for j in range(D - 1):
            _issue(jnp.int32(j))

    j_new = i + D - 1
    b_new = _block_of(j_new)
    @pl.when(jnp.logical_and(_need_fetch(j_new),
                             jnp.logical_or(j_new < R, b_new < CLO - D)))
    def _prefetch():
        _issue(b_new)

    k_early = i - (R + D)

    @pl.when(jnp.logical_and(k_early >= 0, k_early < D))
    def _prefetch_early():
        _issue(CLO - 1 - k_early)

    @pl.when(_need_fetch(i))
    def _wait():
        pltpu.make_async_copy(
            adj_ref.at[pl.ds(b * BLK, BLK), :],
            buf.at[slot],
            sems.at[slot],
        ).wait()

    in_cache = jnp.logical_and(b >= CLO, b < R - D)

    @pl.when(jnp.logical_and(i < R, in_cache))
    def _fill_cache():
        cache[b - CLO] = buf[slot].astype(jnp.bfloat16)

    @pl.when(jnp.logical_not(in_cache))
    def _y_stream():
        y = jnp.dot(buf[slot].astype(jnp.bfloat16), hwb[...],
                    preferred_element_type=jnp.float32)
        ybuf[pl.ds(b * BLK, BLK), :] = y + b1_ref[0]

    @pl.when(in_cache)
    def _y_cached():
        y = jnp.dot(cache[b - CLO], hwb[...],
                    preferred_element_type=jnp.float32)
        ybuf[pl.ds(b * BLK, BLK), :] = y + b1_ref[0]

    @pl.when(jax.lax.rem(i, R) == R - 1)
    def _finish_layer():
        z = _bn_relu(ybuf[...], gm_ref[0], bm_ref[0])
        z2 = jnp.dot(z, W2_ref[0], preferred_element_type=jnp.float32)
        hbuf[...] = _bn_relu(z2 + b2_ref[0], go_ref[0], bo_ref[0])

    @pl.when(i == R - 1)
    def _next_hw():
        hwb[...] = jnp.dot(hbuf[...], W1_ref[0],
                           preferred_element_type=jnp.float32
                           ).astype(jnp.bfloat16)

    @pl.when(i == 2 * R - 1)
    def _head():
        h = hbuf[...]
        hp = jnp.dot(gp_ref[...], h, preferred_element_type=jnp.float32)

        def _gather(j, _):
            idx = cand_ref[j, 0]
            cfbuf[pl.ds(j, 1), :] = hbuf[pl.ds(idx, 1), :]
            return 0
        jax.lax.fori_loop(0, NJ, _gather, 0)
        cf = cfbuf[...]
        Wq0 = Wq0_ref[...]
        x = jnp.dot(cf, Wq0[:HID], preferred_element_type=jnp.float32)
        x = x + jnp.dot(hp, Wq0[HID:], preferred_element_type=jnp.float32)
        x = jnp.maximum(x + bq0_ref[...], 0.0)
        x = jnp.maximum(jnp.dot(x, Wq1_ref[...],
                                preferred_element_type=jnp.float32)
                        + bq1_ref[...], 0.0)
        x = jnp.maximum(jnp.dot(x, Wq2_ref[...],
                                preferred_element_type=jnp.float32)
                        + bq2_ref[...], 0.0)
        q_ref[...] = jnp.dot(x, Wq3_ref[...],
                             preferred_element_type=jnp.float32) + bq3_ref[...]


def kernel(adj, features, candidate, graph_pool, action_mask,
           W1_0, b1_0, gm_0, bm_0, W2_0, b2_0, go_0, bo_0,
           W1_1, b1_1, gm_1, bm_1, W2_1, b2_1, go_1, bo_1,
           Wq0, bq0, Wq1, bq1, Wq2, bq2, Wq3, bq3):
    fpad = jnp.pad(features, ((0, 0), (0, HID - features.shape[1]))
                   ).astype(jnp.bfloat16)
    W1s = jnp.stack([jnp.pad(W1_0, ((0, HID - W1_0.shape[0]), (0, 0))), W1_1])
    W2s = jnp.stack([W2_0, W2_1])
    b1s = jnp.stack([b1_0, b1_1]).reshape(2, 1, HID)
    gms = jnp.stack([gm_0, gm_1]).reshape(2, 1, HID)
    bms = jnp.stack([bm_0, bm_1]).reshape(2, 1, HID)
    b2s = jnp.stack([b2_0, b2_1]).reshape(2, 1, HID)
    gos = jnp.stack([go_0, go_1]).reshape(2, 1, HID)
    bos = jnp.stack([bo_0, bo_1]).reshape(2, 1, HID)
    cand = candidate.reshape(NJ, 1)

    full = lambda shape: pl.BlockSpec(shape, lambda i: (0,) * len(shape))
    layer3 = lambda shape: pl.BlockSpec((1,) + shape, lambda i: (i // R, 0, 0))
    w1spec = pl.BlockSpec((1, HID, HID),
                          lambda i: (jnp.minimum((i + 1) // R, 1), 0, 0))

    q = pl.pallas_call(
        _body,
        grid=(2 * R,),
        in_specs=[
            pl.BlockSpec(memory_space=pl.ANY),
            full((N, HID)),
            w1spec,
            layer3((1, HID)),
            layer3((1, HID)),
            layer3((1, HID)),
            layer3((HID, HID)),
            layer3((1, HID)),
            layer3((1, HID)),
            layer3((1, HID)),
            full((1, N)),
            pl.BlockSpec(memory_space=pltpu.SMEM),
            full((2 * HID, HQ)), full((1, HQ)),
            full((HQ, HQ)), full((1, HQ)),
            full((HQ, HQ)), full((1, HQ)),
            full((HQ, NQ)), full((1, NQ)),
        ],
        out_specs=pl.BlockSpec((NJ, NQ), lambda i: (0, 0)),
        out_shape=jax.ShapeDtypeStruct((NJ, NQ), jnp.float32),
        scratch_shapes=[
            pltpu.VMEM((N, HID), jnp.float32),
            pltpu.VMEM((N, HID), jnp.float32),
            pltpu.VMEM((N, HID), jnp.bfloat16),
            pltpu.VMEM((NJ, HID), jnp.float32),
            pltpu.VMEM((D, BLK, N), jnp.float32),
            pltpu.VMEM((CB, BLK, N), jnp.bfloat16),
            pltpu.SemaphoreType.DMA((D,)),
        ],
        compiler_params=pltpu.CompilerParams(
            dimension_semantics=("arbitrary",),
            vmem_limit_bytes=64 * 1024 * 1024),
    )(adj, fpad, W1s, b1s, gms, bms, W2s, b2s, gos, bos,
      graph_pool, cand, Wq0, bq0.reshape(1, HQ), Wq1, bq1.reshape(1, HQ),
      Wq2, bq2.reshape(1, HQ), Wq3, bq3.reshape(1, NQ))
    return q.reshape(1, NJ, NQ)


# speedup vs baseline: 1.0267x; 1.0267x over previous
import jax
import jax.numpy as jnp
from jax.experimental import pallas as pl
from jax.experimental.pallas import tpu as pltpu

N = 10000
HID = 64
NJ = 100
NQ = 32
HQ = 256
BLK = 200
R = N // BLK
D = 3
CB = 5
CLO = R - D - CB


def _block_of(j):
    phase1 = j < R
    s = jnp.where(phase1, j, j - R)
    return jnp.where(phase1, s, R - 1 - s)


def _need_fetch(j):
    return jnp.logical_and(j < 2 * R,
                           jnp.logical_or(j < R, _block_of(j) < CLO))


def _bn_relu(x, g, b):
    m = jnp.mean(x, axis=0, keepdims=True)
    d = x - m
    v = jnp.mean(d * d, axis=0, keepdims=True)
    return jnp.maximum(d / jnp.sqrt(v + 1e-5) * g + b, 0.0)


def _body(adj_ref, f_ref, W1_ref, b1_ref, gm_ref, bm_ref, W2_ref, b2_ref,
          go_ref, bo_ref, gp_ref, cand_ref, Wq0_ref, bq0_ref, Wq1_ref,
          bq1_ref, Wq2_ref, bq2_ref, Wq3_ref, bq3_ref, q_ref,
          ybuf, hwb, cfbuf, buf, cache, sems):
    i = pl.program_id(0)
    b = _block_of(i)
    slot = jax.lax.rem(b, D)

    def _issue(blk):
        pltpu.make_async_copy(
            adj_ref.at[pl.ds(blk * BLK, BLK), :],
            buf.at[jax.lax.rem(blk, D)],
            sems.at[jax.lax.rem(blk, D)],
        ).start()

    @pl.when(i == 0)
    def _init():
        hwb[...] = jnp.dot(f_ref[...], W1_ref[0].astype(jnp.bfloat16),
                           preferred_element_type=jnp.float32
                           ).astype(jnp.bfloat16)
        for j in range(D - 1):
            _issue(jnp.int32(j))

    j_new = i + D - 1
    b_new = _block_of(j_new)
    @pl.when(jnp.logical_and(_need_fetch(j_new),
                             jnp.logical_or(j_new < R, b_new < CLO - D)))
    def _prefetch():
        _issue(b_new)

    k_early = i - (R + D)

    @pl.when(jnp.logical_and(k_early >= 0, k_early < D))
    def _prefetch_early():
        _issue(CLO - 1 - k_early)

    @pl.when(_need_fetch(i))
    def _wait():
        pltpu.make_async_copy(
            adj_ref.at[pl.ds(b * BLK, BLK), :],
            buf.at[slot],
            sems.at[slot],
        ).wait()

    in_cache = jnp.logical_and(b >= CLO, b < R - D)

    @pl.when(jnp.logical_and(i < R, in_cache))
    def _fill_cache():
        cache[b - CLO] = buf[slot].astype(jnp.bfloat16)

    @pl.when(jnp.logical_not(in_cache))
    def _y_stream():
        y = jnp.dot(buf[slot].astype(jnp.bfloat16), hwb[...],
                    preferred_element_type=jnp.float32)
        ybuf[pl.ds(b * BLK, BLK), :] = y + b1_ref[0]

    @pl.when(in_cache)
    def _y_cached():
        y = jnp.dot(cache[b - CLO], hwb[...],
                    preferred_element_type=jnp.float32)
        ybuf[pl.ds(b * BLK, BLK), :] = y + b1_ref[0]

    def _finish_layer():
        z = _bn_relu(ybuf[...], gm_ref[0], bm_ref[0])
        z2 = jnp.dot(z, W2_ref[0], preferred_element_type=jnp.float32)
        return _bn_relu(z2 + b2_ref[0], go_ref[0], bo_ref[0])

    @pl.when(i == R - 1)
    def _next_hw():
        hwb[...] = jnp.dot(_finish_layer(), W1_ref[0],
                           preferred_element_type=jnp.float32
                           ).astype(jnp.bfloat16)

    @pl.when(i == 2 * R - 1)
    def _head():
        h = _finish_layer()
        ybuf[...] = h
        hp = jnp.dot(gp_ref[...], h, preferred_element_type=jnp.float32)

        def _gather(j, _):
            idx = cand_ref[j, 0]
            cfbuf[pl.ds(j, 1), :] = ybuf[pl.ds(idx, 1), :]
            return 0
        jax.lax.fori_loop(0, NJ, _gather, 0)
        cf = cfbuf[...]
        Wq0 = Wq0_ref[...]
        x = jnp.dot(cf, Wq0[:HID], preferred_element_type=jnp.float32)
        x = x + jnp.dot(hp, Wq0[HID:], preferred_element_type=jnp.float32)
        x = jnp.maximum(x + bq0_ref[...], 0.0)
        x = jnp.maximum(jnp.dot(x, Wq1_ref[...],
                                preferred_element_type=jnp.float32)
                        + bq1_ref[...], 0.0)
        x = jnp.maximum(jnp.dot(x, Wq2_ref[...],
                                preferred_element_type=jnp.float32)
                        + bq2_ref[...], 0.0)
        q_ref[...] = jnp.dot(x, Wq3_ref[...],
                             preferred_element_type=jnp.float32) + bq3_ref[...]


def kernel(adj, features, candidate, graph_pool, action_mask,
           W1_0, b1_0, gm_0, bm_0, W2_0, b2_0, go_0, bo_0,
           W1_1, b1_1, gm_1, bm_1, W2_1, b2_1, go_1, bo_1,
           Wq0, bq0, Wq1, bq1, Wq2, bq2, Wq3, bq3):
    fpad = jnp.pad(features, ((0, 0), (0, HID - features.shape[1]))
                   ).astype(jnp.bfloat16)
    W1s = jnp.stack([jnp.pad(W1_0, ((0, HID - W1_0.shape[0]), (0, 0))), W1_1])
    W2s = jnp.stack([W2_0, W2_1])
    b1s = jnp.stack([b1_0, b1_1]).reshape(2, 1, HID)
    gms = jnp.stack([gm_0, gm_1]).reshape(2, 1, HID)
    bms = jnp.stack([bm_0, bm_1]).reshape(2, 1, HID)
    b2s = jnp.stack([b2_0, b2_1]).reshape(2, 1, HID)
    gos = jnp.stack([go_0, go_1]).reshape(2, 1, HID)
    bos = jnp.stack([bo_0, bo_1]).reshape(2, 1, HID)
    cand = candidate.reshape(NJ, 1)

    full = lambda shape: pl.BlockSpec(shape, lambda i: (0,) * len(shape))
    layer3 = lambda shape: pl.BlockSpec((1,) + shape, lambda i: (i // R, 0, 0))
    w1spec = pl.BlockSpec((1, HID, HID),
                          lambda i: (jnp.minimum((i + 1) // R, 1), 0, 0))

    q = pl.pallas_call(
        _body,
        grid=(2 * R,),
        in_specs=[
            pl.BlockSpec(memory_space=pl.ANY),
            full((N, HID)),
            w1spec,
            layer3((1, HID)),
            layer3((1, HID)),
            layer3((1, HID)),
            layer3((HID, HID)),
            layer3((1, HID)),
            layer3((1, HID)),
            layer3((1, HID)),
            full((1, N)),
            pl.BlockSpec(memory_space=pltpu.SMEM),
            full((2 * HID, HQ)), full((1, HQ)),
            full((HQ, HQ)), full((1, HQ)),
            full((HQ, HQ)), full((1, HQ)),
            full((HQ, NQ)), full((1, NQ)),
        ],
        out_specs=pl.BlockSpec((NJ, NQ), lambda i: (0, 0)),
        out_shape=jax.ShapeDtypeStruct((NJ, NQ), jnp.float32),
        scratch_shapes=[
            pltpu.VMEM((N, HID), jnp.float32),
            pltpu.VMEM((N, HID), jnp.bfloat16),
            pltpu.VMEM((NJ, HID), jnp.float32),
            pltpu.VMEM((D, BLK, N), jnp.float32),
            pltpu.VMEM((CB, BLK, N), jnp.bfloat16),
            pltpu.SemaphoreType.DMA((D,)),
        ],
        compiler_params=pltpu.CompilerParams(
            dimension_semantics=("arbitrary",),
            vmem_limit_bytes=64 * 1024 * 1024),
    )(adj, fpad, W1s, b1s, gms, bms, W2s, b2s, gos, bos,
      graph_pool, cand, Wq0, bq0.reshape(1, HQ), Wq1, bq1.reshape(1, HQ),
      Wq2, bq2.reshape(1, HQ), Wq3, bq3.reshape(1, NQ))
    return q.reshape(1, NJ, NQ)
